# Initial kernel scaffold; baseline (speedup 1.0000x reference)
import jax
import jax.numpy as jnp
from jax.experimental import pallas as pl


def kernel(x, edge_index, W1, b1, W2, b2):
    raise NotImplementedError("write your pallas kernel here")


# speedup vs baseline: 9.2420x; 9.2420x over previous
import functools

import jax
import jax.numpy as jnp
from jax import lax
from jax.experimental import pallas as pl
from jax.experimental.pallas import tpu as pltpu
from jax.experimental.pallas import tpu_sc as plsc

N = 10000
D = 128
NP = 10240
NW = 32
C = 128
K = 79
EP = NW * K * C
DUMP = N + 100
RPT = NP // 16

@functools.lru_cache(maxsize=1)
def _sc_kernels():
    mesh = plsc.VectorSubcoreMesh(core_axis_name="c", subcore_axis_name="s")

    @functools.partial(
        pl.kernel,
        out_type=jax.ShapeDtypeStruct((2, NP, D), jnp.float32),
        mesh=mesh,
        scratch_types=[
            pltpu.VMEM((K, C), jnp.int32),
            pltpu.VMEM((K, C), jnp.int32),
            pltpu.VMEM((C, D), jnp.float32),
            pltpu.VMEM_SHARED((NP, D), jnp.float32),
            pltpu.SemaphoreType.DMA,
        ],
    )
    def spmm_sc(hs_hbm, src_hbm, dst_hbm, zerosd_hbm, out_hbm,
                src_v, dst_v, rows_v, acc, sem):
        cid = lax.axis_index("c")
        sid = lax.axis_index("s")
        wid = sid * 2 + cid
        pltpu.sync_copy(zerosd_hbm.at[pl.ds(sid * RPT, RPT)],
                        acc.at[pl.ds(sid * RPT, RPT)])
        pltpu.sync_copy(src_hbm.at[wid], src_v)
        pltpu.sync_copy(dst_hbm.at[wid], dst_v)
        plsc.subcore_barrier()

        def body(j, carry):
            pltpu.async_copy(hs_hbm.at[src_v.at[j]], rows_v, sem).wait()
            pltpu.sync_copy(rows_v, acc.at[dst_v.at[j]], add=True)
            return carry

        lax.fori_loop(0, K, body, 0)
        plsc.subcore_barrier()
        pltpu.sync_copy(acc.at[pl.ds(sid * RPT, RPT)],
                        out_hbm.at[cid, pl.ds(sid * RPT, RPT)])

    return spmm_sc


_GB = 10
_BN = N // _GB


def _dinv_block(degp):
    deg = degp[0, :, 0:1] + degp[1, :, 0:1] + 1.0
    return lax.rsqrt(deg)


def _mm_tc(x_ref, w_ref, o_ref):
    o_ref[...] = jnp.dot(x_ref[...], w_ref[...],
                         preferred_element_type=jnp.float32)


def _scale_tc(h_ref, degp_ref, o_ref):
    o_ref[...] = h_ref[...] * _dinv_block(degp_ref[...])


def _layer2_tc(p_ref, hs_ref, degp_ref, w_ref, b_ref, o_ref):
    dinv = _dinv_block(degp_ref[...])
    z = (p_ref[0] + p_ref[1] + hs_ref[...]) * dinv + b_ref[...]
    z = jnp.maximum(z, 0.0)
    o_ref[...] = jnp.dot(z, w_ref[...],
                         preferred_element_type=jnp.float32) * dinv


def _final_tc(q_ref, hs_ref, degp_ref, b_ref, o_ref):
    dinv = _dinv_block(degp_ref[...])
    o_ref[...] = (q_ref[0] + q_ref[1] + hs_ref[...]) * dinv + b_ref[...]


_spec_rows = pl.BlockSpec((_BN, D), lambda i: (i, 0))
_spec_w = pl.BlockSpec((D, D), lambda i: (0, 0))
_spec_b = pl.BlockSpec((1, D), lambda i: (0, 0))
_spec_p = pl.BlockSpec((2, _BN, D), lambda i: (0, i, 0))
_spec_deg = pl.BlockSpec((2, _BN, D), lambda i: (0, i, 0))
_out_rows = jax.ShapeDtypeStruct((N, D), jnp.float32)


def kernel(x, edge_index, W1, b1, W2, b2):
    src = edge_index[0].astype(jnp.int32)
    dst = edge_index[1].astype(jnp.int32)
    e = src.shape[0]
    pad = EP - e
    srcw = jnp.concatenate(
        [src, jnp.zeros((pad,), jnp.int32)]).reshape(NW, K, C)
    dstw = jnp.concatenate(
        [dst, jnp.full((pad,), DUMP, jnp.int32)]).reshape(NW, K, C)
    zerosd = jnp.zeros((NP, D), jnp.float32)
    onesn = jnp.ones((N, D), jnp.float32)
    b1r = b1.reshape(1, D)
    b2r = b2.reshape(1, D)

    spmm_sc = _sc_kernels()
    degp = spmm_sc(onesn, srcw, dstw, zerosd)

    h1 = pl.pallas_call(
        _mm_tc, grid=(_GB,),
        in_specs=[_spec_rows, _spec_w], out_specs=_spec_rows,
        out_shape=_out_rows)(x, W1)

    hs1 = pl.pallas_call(
        _scale_tc, grid=(_GB,),
        in_specs=[_spec_rows, _spec_deg], out_specs=_spec_rows,
        out_shape=_out_rows)(h1, degp)

    p = spmm_sc(hs1, srcw, dstw, zerosd)

    hs2 = pl.pallas_call(
        _layer2_tc, grid=(_GB,),
        in_specs=[_spec_p, _spec_rows, _spec_deg, _spec_w, _spec_b],
        out_specs=_spec_rows, out_shape=_out_rows)(p, hs1, degp, W2, b1r)

    q = spmm_sc(hs2, srcw, dstw, zerosd)

    out = pl.pallas_call(
        _final_tc, grid=(_GB,),
        in_specs=[_spec_p, _spec_rows, _spec_deg, _spec_b],
        out_specs=_spec_rows, out_shape=_out_rows)(q, hs2, degp, b2r)

    return out
